# baseline (speedup 1.0000x reference)
import jax
import jax.numpy as jnp
from jax import lax
from jax.experimental import pallas as pl
from jax.experimental.pallas import tpu as pltpu
from jax.experimental.pallas import tpu_sc as plsc

N = 10000
E = 320000
D = 128
NP = 10240
EP = 327680
RPT = NP // 16
HR = NP // 128
HRT = HR // 16
EPT = EP // 16
CE = 128
EPC = EP // 2
EPCT = EPC // 16
NCE = EPCT // CE
SB = 1024
CPS = SB // CE
CZ = 32
NCZ = RPT // CZ


def _rsqrt16(d):
    ii = lax.bitcast_convert_type(d, jnp.int32)
    ii = jnp.int32(0x5F3759DF) - (ii >> 1)
    y = lax.bitcast_convert_type(ii, jnp.float32)
    half = jnp.float32(0.5) * d
    y = y * (jnp.float32(1.5) - half * y * y)
    y = y * (jnp.float32(1.5) - half * y * y)
    y = y * (jnp.float32(1.5) - half * y * y)
    return y


def _main_body(x_hbm, src_hbm, dst_hbm, part_hbm, xs_hbm,
               deg_sh, acc_sh, zdeg, iden, dbuf, ibuf, xrow,
               ssrcb, sdstb, sidxa, didxa, sidxb, didxb,
               rowsa, rowsb, abuf, sema, semb, ssema, ssemb):
    hist = rowsa
    zacc = abuf
    c = lax.axis_index("c")
    s = lax.axis_index("s")
    f0 = jnp.float32(0.0)

    def fill_zacc(i, _):
        for k in range(D // 16):
            zacc[i, pl.ds(k * 16, 16)] = jnp.full((16,), f0)
        return _
    lax.fori_loop(0, CZ, fill_zacc, None)

    def fill_zdeg(i, _):
        for k in range(D // 16):
            zdeg[i, pl.ds(k * 16, 16)] = jnp.full((16,), f0)
        return _
    lax.fori_loop(0, HRT, fill_zdeg, None)

    def fill_iden(k, _):
        iden[pl.ds(k * 16, 16)] = lax.iota(jnp.int32, 16) + k * 16
        return _
    lax.fori_loop(0, HR // 16, fill_iden, None)

    def fill_hist(i, _):
        for k in range(D // 16):
            hist[i, pl.ds(k * 16, 16)] = jnp.full((16,), f0)
        return _
    lax.fori_loop(0, HR, fill_hist, None)

    pltpu.sync_copy(zdeg, deg_sh.at[pl.ds(s * HRT, HRT), :])

    def zero_acc(j, _):
        pltpu.sync_copy(zacc, acc_sh.at[pl.ds(s * RPT + j * CZ, CZ), :])
        return _
    lax.fori_loop(0, NCZ, zero_acc, None)
    plsc.subcore_barrier()

    one16 = jnp.full((16,), jnp.float32(1.0))

    def deg_block(q, _):
        pltpu.sync_copy(src_hbm.at[pl.ds(s * EPT + q * SB, SB)], ssrcb)

        def deg_step(g, _):
            n = ssrcb[pl.ds(g * 16, 16)]
            plsc.addupdate_scatter(hist, [n >> 7, n & 127], one16)
            return _
        lax.fori_loop(0, SB // 16, deg_step, None)
        return _
    lax.fori_loop(0, EPT // SB, deg_block, None)
    pltpu.sync_copy(hist.at[pl.ds(0, HR), :], deg_sh.at[iden], add=True)
    plsc.subcore_barrier()

    pltpu.sync_copy(deg_sh.at[pl.ds(s * HRT, HRT), :], dbuf)

    def inv_step(i, _):
        r = i // 8
        k = i % 8
        d = jnp.maximum(dbuf[r, pl.ds(k * 16, 16)], jnp.float32(1.0))
        ibuf[pl.ds(i * 16, 16)] = _rsqrt16(d)
        return _
    lax.fori_loop(0, RPT // 16, inv_step, None)

    def pre_chunk(j, _):
        r0 = s * RPT + j * CZ
        pltpu.sync_copy(x_hbm.at[pl.ds(r0, CZ), :], xrow)

        def pre_row(i, _):
            idxv = jnp.zeros((16,), jnp.int32) + (j * CZ + i)
            sc = plsc.load_gather(ibuf, [idxv])
            for k in range(D // 16):
                xrow[i, pl.ds(k * 16, 16)] = xrow[i, pl.ds(k * 16, 16)] * sc
            return _
        lax.fori_loop(0, CZ, pre_row, None)
        pltpu.sync_copy(xrow, xs_hbm.at[pl.ds(c * NP + r0, CZ), :])
        return _
    lax.fori_loop(0, NCZ, pre_chunk, None)
    plsc.subcore_barrier()

    base = c * NP

    def handle(j, sidxp, didxp, rowsp, semp, ssemp):
        @pl.when(j >= 2)
        def _():
            pltpu.make_async_copy(rowsp, acc_sh.at[didxp], ssemp).wait()

        @pl.when(j % CPS == 0)
        def _():
            e0 = c * EPC + s * EPCT + (j // CPS) * SB
            pltpu.sync_copy(src_hbm.at[pl.ds(e0, SB)], ssrcb)
            pltpu.sync_copy(dst_hbm.at[pl.ds(e0, SB)], sdstb)

        off = (j % CPS) * CE
        for k in range(CE // 16):
            sidxp[pl.ds(k * 16, 16)] = ssrcb[pl.ds(off + k * 16, 16)] + base
            didxp[pl.ds(k * 16, 16)] = sdstb[pl.ds(off + k * 16, 16)]
        pltpu.async_copy(xs_hbm.at[sidxp], rowsp, semp)

    def finish(sidxp, didxp, rowsp, semp, ssemp):
        pltpu.make_async_copy(xs_hbm.at[sidxp], rowsp, semp).wait()
        pltpu.async_copy(rowsp, acc_sh.at[didxp], ssemp, add=True)

    handle(0, sidxa, didxa, rowsa, sema, ssema)

    def edge_pair(t, _):
        handle(2 * t + 1, sidxb, didxb, rowsb, semb, ssemb)
        finish(sidxa, didxa, rowsa, sema, ssema)
        handle(2 * t + 2, sidxa, didxa, rowsa, sema, ssema)
        finish(sidxb, didxb, rowsb, semb, ssemb)
        return _
    lax.fori_loop(0, (NCE - 1) // 2, edge_pair, None)
    if NCE % 2 == 1:
        finish(sidxa, didxa, rowsa, sema, ssema)
    else:
        handle(NCE - 1, sidxb, didxb, rowsb, semb, ssemb)
        finish(sidxa, didxa, rowsa, sema, ssema)
        finish(sidxb, didxb, rowsb, semb, ssemb)
    pltpu.make_async_copy(rowsb, acc_sh.at[didxb], ssemb).wait()
    pltpu.make_async_copy(rowsa, acc_sh.at[didxa], ssema).wait()
    plsc.subcore_barrier()

    def out_chunk(j, _):
        r0 = s * RPT + j * CZ
        pltpu.sync_copy(acc_sh.at[pl.ds(r0, CZ), :], abuf)

        def out_row(i, _):
            idxv = jnp.zeros((16,), jnp.int32) + (j * CZ + i)
            sc = plsc.load_gather(ibuf, [idxv])
            for k in range(D // 16):
                abuf[i, pl.ds(k * 16, 16)] = abuf[i, pl.ds(k * 16, 16)] * sc
            return _
        lax.fori_loop(0, CZ, out_row, None)
        pltpu.sync_copy(abuf, part_hbm.at[c, pl.ds(r0, CZ), :])
        return _
    lax.fori_loop(0, NCZ, out_chunk, None)


def _sum_tc_body(p_ref, o_ref):
    o_ref[...] = p_ref[0] + p_ref[1]


@jax.jit
def _gcn_sc(x_pad, src, dst):
    mesh = plsc.VectorSubcoreMesh(core_axis_name="c", subcore_axis_name="s")
    ka = pl.kernel(
        _main_body,
        out_type=(
            jax.ShapeDtypeStruct((2, NP, D), jnp.float32),
            jax.ShapeDtypeStruct((2 * NP, D), jnp.float32),
        ),
        mesh=mesh,
        compiler_params=pltpu.CompilerParams(needs_layout_passes=False),
        scratch_types=[
            pltpu.VMEM_SHARED((HR, D), jnp.float32),
            pltpu.VMEM_SHARED((NP, D), jnp.float32),
            pltpu.VMEM((HRT, D), jnp.float32),
            pltpu.VMEM((HR,), jnp.int32),
            pltpu.VMEM((HRT, D), jnp.float32),
            pltpu.VMEM((RPT,), jnp.float32),
            pltpu.VMEM((CZ, D), jnp.float32),
            pltpu.VMEM((SB,), jnp.int32),
            pltpu.VMEM((SB,), jnp.int32),
            pltpu.VMEM((CE,), jnp.int32),
            pltpu.VMEM((CE,), jnp.int32),
            pltpu.VMEM((CE,), jnp.int32),
            pltpu.VMEM((CE,), jnp.int32),
            pltpu.VMEM((CE, D), jnp.float32),
            pltpu.VMEM((CE, D), jnp.float32),
            pltpu.VMEM((CZ, D), jnp.float32),
            pltpu.SemaphoreType.DMA,
            pltpu.SemaphoreType.DMA,
            pltpu.SemaphoreType.DMA,
            pltpu.SemaphoreType.DMA,
        ],
    )
    part, _ = ka(x_pad, src, dst)
    return pl.pallas_call(
        _sum_tc_body,
        grid=(NP // 512,),
        in_specs=[pl.BlockSpec((2, 512, D), lambda i: (0, i, 0))],
        out_specs=pl.BlockSpec((512, D), lambda i: (i, 0)),
        out_shape=jax.ShapeDtypeStruct((NP, D), jnp.float32),
    )(part)


def kernel(x, edge_index):
    ei = edge_index.astype(jnp.int32)
    ei = jnp.pad(ei, ((0, 0), (0, EP - E)), constant_values=NP - 1)
    x_pad = jnp.pad(x, ((0, NP - N), (0, 0)))
    out = _gcn_sc(x_pad, ei[0], ei[1])
    return out[:N]


# speedup vs baseline: 1.0068x; 1.0068x over previous
import jax
import jax.numpy as jnp
from jax import lax
from jax.experimental import pallas as pl
from jax.experimental.pallas import tpu as pltpu
from jax.experimental.pallas import tpu_sc as plsc

N = 10000
E = 320000
D = 128
NP = 10240
EP = 327680
RPT = NP // 16
HR = NP // 128
HRT = HR // 16
EPT = EP // 16
CE = 64
EPC = EP // 2
EPCT = EPC // 16
NCE = EPCT // CE
SB = 2048
CPS = SB // CE
CZ = 32
NCZ = RPT // CZ


def _rsqrt16(d):
    ii = lax.bitcast_convert_type(d, jnp.int32)
    ii = jnp.int32(0x5F3759DF) - (ii >> 1)
    y = lax.bitcast_convert_type(ii, jnp.float32)
    half = jnp.float32(0.5) * d
    y = y * (jnp.float32(1.5) - half * y * y)
    y = y * (jnp.float32(1.5) - half * y * y)
    y = y * (jnp.float32(1.5) - half * y * y)
    return y


def _main_body(x_hbm, src_hbm, dst_hbm, part_hbm, xs_hbm,
               deg_sh, acc_sh, zdeg, iden, dbuf, ibuf, xrow,
               ssrcb, sdstb, sidxa, didxa, sidxb, didxb,
               rowsa, rowsb, abuf, sema, semb, ssema, ssemb):
    hist = rowsa
    zacc = abuf
    c = lax.axis_index("c")
    s = lax.axis_index("s")
    f0 = jnp.float32(0.0)

    def fill_zacc(i, _):
        for k in range(D // 16):
            zacc[i, pl.ds(k * 16, 16)] = jnp.full((16,), f0)
        return _
    lax.fori_loop(0, CZ, fill_zacc, None)

    def fill_zdeg(i, _):
        for k in range(D // 16):
            zdeg[i, pl.ds(k * 16, 16)] = jnp.full((16,), f0)
        return _
    lax.fori_loop(0, HRT, fill_zdeg, None)

    def fill_iden(k, _):
        iden[pl.ds(k * 16, 16)] = lax.iota(jnp.int32, 16) + k * 16
        return _
    lax.fori_loop(0, HR // 16, fill_iden, None)

    def fill_hist(i, _):
        for k in range(D // 16):
            hist[i, pl.ds(k * 16, 16)] = jnp.full((16,), f0)
        return _
    lax.fori_loop(0, HR, fill_hist, None)

    pltpu.sync_copy(zdeg, deg_sh.at[pl.ds(s * HRT, HRT), :])

    def zero_acc(j, _):
        pltpu.sync_copy(zacc, acc_sh.at[pl.ds(s * RPT + j * CZ, CZ), :])
        return _
    lax.fori_loop(0, NCZ, zero_acc, None)
    plsc.subcore_barrier()

    one16 = jnp.full((16,), jnp.float32(1.0))

    def deg_block(q, _):
        pltpu.sync_copy(src_hbm.at[pl.ds(s * EPT + q * SB, SB)], ssrcb)

        def deg_step(g, _):
            n = ssrcb[pl.ds(g * 16, 16)]
            plsc.addupdate_scatter(hist, [n >> 7, n & 127], one16)
            return _
        lax.fori_loop(0, SB // 16, deg_step, None)
        return _
    lax.fori_loop(0, EPT // SB, deg_block, None)
    pltpu.sync_copy(hist.at[pl.ds(0, HR), :], deg_sh.at[iden], add=True)
    plsc.subcore_barrier()

    pltpu.sync_copy(deg_sh.at[pl.ds(s * HRT, HRT), :], dbuf)

    def inv_step(i, _):
        r = i // 8
        k = i % 8
        d = jnp.maximum(dbuf[r, pl.ds(k * 16, 16)], jnp.float32(1.0))
        ibuf[pl.ds(i * 16, 16)] = _rsqrt16(d)
        return _
    lax.fori_loop(0, RPT // 16, inv_step, None)

    def pre_chunk(j, _):
        r0 = s * RPT + j * CZ
        pltpu.sync_copy(x_hbm.at[pl.ds(r0, CZ), :], xrow)

        def pre_row(i, _):
            idxv = jnp.zeros((16,), jnp.int32) + (j * CZ + i)
            sc = plsc.load_gather(ibuf, [idxv])
            for k in range(D // 16):
                xrow[i, pl.ds(k * 16, 16)] = xrow[i, pl.ds(k * 16, 16)] * sc
            return _
        lax.fori_loop(0, CZ, pre_row, None)
        pltpu.sync_copy(xrow, xs_hbm.at[pl.ds(c * NP + r0, CZ), :])
        return _
    lax.fori_loop(0, NCZ, pre_chunk, None)
    plsc.subcore_barrier()

    base = c * NP

    def handle(j, sidxp, didxp, rowsp, semp, ssemp):
        @pl.when(j >= 2)
        def _():
            pltpu.make_async_copy(rowsp, acc_sh.at[didxp], ssemp).wait()

        @pl.when(j % CPS == 0)
        def _():
            e0 = c * EPC + s * EPCT + (j // CPS) * SB
            pltpu.sync_copy(src_hbm.at[pl.ds(e0, SB)], ssrcb)
            pltpu.sync_copy(dst_hbm.at[pl.ds(e0, SB)], sdstb)

        off = (j % CPS) * CE
        for k in range(CE // 16):
            sidxp[pl.ds(k * 16, 16)] = ssrcb[pl.ds(off + k * 16, 16)] + base
            didxp[pl.ds(k * 16, 16)] = sdstb[pl.ds(off + k * 16, 16)]
        pltpu.async_copy(xs_hbm.at[sidxp], rowsp, semp)

    def finish(sidxp, didxp, rowsp, semp, ssemp):
        pltpu.make_async_copy(xs_hbm.at[sidxp], rowsp, semp).wait()
        pltpu.async_copy(rowsp, acc_sh.at[didxp], ssemp, add=True)

    handle(0, sidxa, didxa, rowsa, sema, ssema)

    def edge_pair(t, _):
        handle(2 * t + 1, sidxb, didxb, rowsb, semb, ssemb)
        finish(sidxa, didxa, rowsa, sema, ssema)
        handle(2 * t + 2, sidxa, didxa, rowsa, sema, ssema)
        finish(sidxb, didxb, rowsb, semb, ssemb)
        return _
    lax.fori_loop(0, (NCE - 1) // 2, edge_pair, None)
    if NCE % 2 == 1:
        finish(sidxa, didxa, rowsa, sema, ssema)
    else:
        handle(NCE - 1, sidxb, didxb, rowsb, semb, ssemb)
        finish(sidxa, didxa, rowsa, sema, ssema)
        finish(sidxb, didxb, rowsb, semb, ssemb)
    pltpu.make_async_copy(rowsb, acc_sh.at[didxb], ssemb).wait()
    pltpu.make_async_copy(rowsa, acc_sh.at[didxa], ssema).wait()
    plsc.subcore_barrier()

    def out_chunk(j, _):
        r0 = s * RPT + j * CZ
        pltpu.sync_copy(acc_sh.at[pl.ds(r0, CZ), :], abuf)

        def out_row(i, _):
            idxv = jnp.zeros((16,), jnp.int32) + (j * CZ + i)
            sc = plsc.load_gather(ibuf, [idxv])
            for k in range(D // 16):
                abuf[i, pl.ds(k * 16, 16)] = abuf[i, pl.ds(k * 16, 16)] * sc
            return _
        lax.fori_loop(0, CZ, out_row, None)
        pltpu.sync_copy(abuf, part_hbm.at[c, pl.ds(r0, CZ), :])
        return _
    lax.fori_loop(0, NCZ, out_chunk, None)


def _sum_tc_body(p_ref, o_ref):
    o_ref[...] = p_ref[0] + p_ref[1]


@jax.jit
def _gcn_sc(x_pad, src, dst):
    mesh = plsc.VectorSubcoreMesh(core_axis_name="c", subcore_axis_name="s")
    ka = pl.kernel(
        _main_body,
        out_type=(
            jax.ShapeDtypeStruct((2, NP, D), jnp.float32),
            jax.ShapeDtypeStruct((2 * NP, D), jnp.float32),
        ),
        mesh=mesh,
        compiler_params=pltpu.CompilerParams(needs_layout_passes=False),
        scratch_types=[
            pltpu.VMEM_SHARED((HR, D), jnp.float32),
            pltpu.VMEM_SHARED((NP, D), jnp.float32),
            pltpu.VMEM((HRT, D), jnp.float32),
            pltpu.VMEM((HR,), jnp.int32),
            pltpu.VMEM((HRT, D), jnp.float32),
            pltpu.VMEM((RPT,), jnp.float32),
            pltpu.VMEM((CZ, D), jnp.float32),
            pltpu.VMEM((SB,), jnp.int32),
            pltpu.VMEM((SB,), jnp.int32),
            pltpu.VMEM((CE,), jnp.int32),
            pltpu.VMEM((CE,), jnp.int32),
            pltpu.VMEM((CE,), jnp.int32),
            pltpu.VMEM((CE,), jnp.int32),
            pltpu.VMEM((CE, D), jnp.float32),
            pltpu.VMEM((CE, D), jnp.float32),
            pltpu.VMEM((CZ, D), jnp.float32),
            pltpu.SemaphoreType.DMA,
            pltpu.SemaphoreType.DMA,
            pltpu.SemaphoreType.DMA,
            pltpu.SemaphoreType.DMA,
        ],
    )
    part, _ = ka(x_pad, src, dst)
    return pl.pallas_call(
        _sum_tc_body,
        grid=(NP // 512,),
        in_specs=[pl.BlockSpec((2, 512, D), lambda i: (0, i, 0))],
        out_specs=pl.BlockSpec((512, D), lambda i: (i, 0)),
        out_shape=jax.ShapeDtypeStruct((NP, D), jnp.float32),
    )(part)


def kernel(x, edge_index):
    ei = edge_index.astype(jnp.int32)
    ei = jnp.pad(ei, ((0, 0), (0, EP - E)), constant_values=NP - 1)
    x_pad = jnp.pad(x, ((0, NP - N), (0, 0)))
    out = _gcn_sc(x_pad, ei[0], ei[1])
    return out[:N]


# speedup vs baseline: 2.3342x; 2.3185x over previous
import jax
import jax.numpy as jnp
from jax import lax
from jax.experimental import pallas as pl
from jax.experimental.pallas import tpu as pltpu
from jax.experimental.pallas import tpu_sc as plsc

N = 10000
E = 320000
D = 128
NP = 10240
RPT = NP // 16
HR = NP // 128
HRT = HR // 16
EPT = E // 16
CE = 80
EPC = E // 2
EPCT = EPC // 16
NCE = EPCT // CE
SB = 2000
CPS = SB // CE
CZ = 32
NCZ = RPT // CZ


def _rsqrt16(d):
    ii = lax.bitcast_convert_type(d, jnp.int32)
    ii = jnp.int32(0x5F3759DF) - (ii >> 1)
    y = lax.bitcast_convert_type(ii, jnp.float32)
    half = jnp.float32(0.5) * d
    y = y * (jnp.float32(1.5) - half * y * y)
    y = y * (jnp.float32(1.5) - half * y * y)
    y = y * (jnp.float32(1.5) - half * y * y)
    return y


def _main_body(x_hbm, src_hbm, dst_hbm, part_hbm, xs_hbm,
               deg_sh, acc_sh, zdeg, iden, dbuf, ibuf, xrow,
               ssrcb, sdstb, sidxa, didxa, sidxb, didxb,
               rowsa, rowsb, abuf, sema, semb, ssema, ssemb):
    hist = rowsa
    zacc = abuf
    c = lax.axis_index("c")
    s = lax.axis_index("s")
    f0 = jnp.float32(0.0)

    def fill_zacc(i, _):
        for k in range(D // 16):
            zacc[i, pl.ds(k * 16, 16)] = jnp.full((16,), f0)
        return _
    lax.fori_loop(0, CZ, fill_zacc, None)

    def fill_zdeg(i, _):
        for k in range(D // 16):
            zdeg[i, pl.ds(k * 16, 16)] = jnp.full((16,), f0)
        return _
    lax.fori_loop(0, HRT, fill_zdeg, None)

    def fill_iden(k, _):
        iden[pl.ds(k * 16, 16)] = lax.iota(jnp.int32, 16) + k * 16
        return _
    lax.fori_loop(0, HR // 16, fill_iden, None)

    def fill_hist(i, _):
        for k in range(D // 16):
            hist[i, pl.ds(k * 16, 16)] = jnp.full((16,), f0)
        return _
    lax.fori_loop(0, HR, fill_hist, None)

    pltpu.sync_copy(zdeg, deg_sh.at[pl.ds(s * HRT, HRT), :])

    def zero_acc(j, _):
        pltpu.sync_copy(zacc, acc_sh.at[pl.ds(s * RPT + j * CZ, CZ), :])
        return _
    lax.fori_loop(0, NCZ, zero_acc, None)
    plsc.subcore_barrier()

    one16 = jnp.full((16,), jnp.float32(1.0))

    def deg_block(q, _):
        pltpu.sync_copy(src_hbm.at[pl.ds(s * EPT + q * SB, SB)], ssrcb)

        def deg_step(g, _):
            n = ssrcb[pl.ds(g * 16, 16)]
            plsc.addupdate_scatter(hist, [n >> 7, n & 127], one16)
            return _
        lax.fori_loop(0, SB // 16, deg_step, None)
        return _
    lax.fori_loop(0, EPT // SB, deg_block, None)
    pltpu.sync_copy(hist, deg_sh.at[iden], add=True)
    plsc.subcore_barrier()

    pltpu.sync_copy(deg_sh.at[pl.ds(s * HRT, HRT), :], dbuf)

    def inv_step(i, _):
        r = i // 8
        k = i % 8
        d = jnp.maximum(dbuf[r, pl.ds(k * 16, 16)], jnp.float32(1.0))
        ibuf[pl.ds(i * 16, 16)] = _rsqrt16(d)
        return _
    lax.fori_loop(0, RPT // 16, inv_step, None)

    def pre_chunk(j, _):
        r0 = s * RPT + j * CZ
        pltpu.sync_copy(x_hbm.at[pl.ds(r0, CZ), :], xrow)

        def pre_row(i, _):
            idxv = jnp.zeros((16,), jnp.int32) + (j * CZ + i)
            sc = plsc.load_gather(ibuf, [idxv])
            for k in range(D // 16):
                xrow[i, pl.ds(k * 16, 16)] = xrow[i, pl.ds(k * 16, 16)] * sc
            return _
        lax.fori_loop(0, CZ, pre_row, None)
        pltpu.sync_copy(xrow, xs_hbm.at[pl.ds(c * NP + r0, CZ), :])
        return _
    lax.fori_loop(0, NCZ, pre_chunk, None)
    plsc.subcore_barrier()

    base = c * NP

    def handle(j, sidxp, didxp, rowsp, semp, ssemp):
        @pl.when(j >= 2)
        def _():
            pltpu.make_async_copy(rowsp, acc_sh.at[didxp], ssemp).wait()

        @pl.when(j % CPS == 0)
        def _():
            e0 = c * EPC + s * EPCT + (j // CPS) * SB
            pltpu.sync_copy(src_hbm.at[pl.ds(e0, SB)], ssrcb)
            pltpu.sync_copy(dst_hbm.at[pl.ds(e0, SB)], sdstb)

        off = (j % CPS) * CE
        for k in range(CE // 16):
            sidxp[pl.ds(k * 16, 16)] = ssrcb[pl.ds(off + k * 16, 16)] + base
            didxp[pl.ds(k * 16, 16)] = sdstb[pl.ds(off + k * 16, 16)]
        pltpu.async_copy(xs_hbm.at[sidxp], rowsp, semp)

    def finish(sidxp, didxp, rowsp, semp, ssemp):
        pltpu.make_async_copy(xs_hbm.at[sidxp], rowsp, semp).wait()
        pltpu.async_copy(rowsp, acc_sh.at[didxp], ssemp, add=True)

    handle(0, sidxa, didxa, rowsa, sema, ssema)

    def edge_pair(t, _):
        handle(2 * t + 1, sidxb, didxb, rowsb, semb, ssemb)
        finish(sidxa, didxa, rowsa, sema, ssema)
        handle(2 * t + 2, sidxa, didxa, rowsa, sema, ssema)
        finish(sidxb, didxb, rowsb, semb, ssemb)
        return _
    lax.fori_loop(0, (NCE - 1) // 2, edge_pair, None)
    if NCE % 2 == 1:
        finish(sidxa, didxa, rowsa, sema, ssema)
    else:
        handle(NCE - 1, sidxb, didxb, rowsb, semb, ssemb)
        finish(sidxa, didxa, rowsa, sema, ssema)
        finish(sidxb, didxb, rowsb, semb, ssemb)
    pltpu.make_async_copy(rowsb, acc_sh.at[didxb], ssemb).wait()
    pltpu.make_async_copy(rowsa, acc_sh.at[didxa], ssema).wait()
    plsc.subcore_barrier()

    def out_chunk(j, _):
        r0 = s * RPT + j * CZ
        pltpu.sync_copy(acc_sh.at[pl.ds(r0, CZ), :], abuf)

        def out_row(i, _):
            idxv = jnp.zeros((16,), jnp.int32) + (j * CZ + i)
            sc = plsc.load_gather(ibuf, [idxv])
            for k in range(D // 16):
                abuf[i, pl.ds(k * 16, 16)] = abuf[i, pl.ds(k * 16, 16)] * sc
            return _
        lax.fori_loop(0, CZ, out_row, None)
        pltpu.sync_copy(abuf, part_hbm.at[c, pl.ds(r0, CZ), :])
        return _
    lax.fori_loop(0, NCZ, out_chunk, None)


def _sum_tc_body(p_ref, o_ref):
    o_ref[...] = p_ref[0] + p_ref[1]


@jax.jit
def _gcn_sc(x_pad, src, dst):
    mesh = plsc.VectorSubcoreMesh(core_axis_name="c", subcore_axis_name="s")
    ka = pl.kernel(
        _main_body,
        out_type=(
            jax.ShapeDtypeStruct((2, NP, D), jnp.float32),
            jax.ShapeDtypeStruct((2 * NP, D), jnp.float32),
        ),
        mesh=mesh,
        compiler_params=pltpu.CompilerParams(needs_layout_passes=False),
        scratch_types=[
            pltpu.VMEM_SHARED((HR, D), jnp.float32),
            pltpu.VMEM_SHARED((NP, D), jnp.float32),
            pltpu.VMEM((HRT, D), jnp.float32),
            pltpu.VMEM((HR,), jnp.int32),
            pltpu.VMEM((HRT, D), jnp.float32),
            pltpu.VMEM((RPT,), jnp.float32),
            pltpu.VMEM((CZ, D), jnp.float32),
            pltpu.VMEM((SB,), jnp.int32),
            pltpu.VMEM((SB,), jnp.int32),
            pltpu.VMEM((CE,), jnp.int32),
            pltpu.VMEM((CE,), jnp.int32),
            pltpu.VMEM((CE,), jnp.int32),
            pltpu.VMEM((CE,), jnp.int32),
            pltpu.VMEM((CE, D), jnp.float32),
            pltpu.VMEM((CE, D), jnp.float32),
            pltpu.VMEM((CZ, D), jnp.float32),
            pltpu.SemaphoreType.DMA,
            pltpu.SemaphoreType.DMA,
            pltpu.SemaphoreType.DMA,
            pltpu.SemaphoreType.DMA,
        ],
    )
    part, _ = ka(x_pad, src, dst)
    return pl.pallas_call(
        _sum_tc_body,
        grid=(NP // 512,),
        in_specs=[pl.BlockSpec((2, 512, D), lambda i: (0, i, 0))],
        out_specs=pl.BlockSpec((512, D), lambda i: (i, 0)),
        out_shape=jax.ShapeDtypeStruct((NP, D), jnp.float32),
    )(part)


def kernel(x, edge_index):
    src = edge_index[0].astype(jnp.int32)
    dst = edge_index[1].astype(jnp.int32)
    x_pad = jnp.pad(x, ((0, NP - N), (0, 0)))
    out = _gcn_sc(x_pad, src, dst)
    return out[:N]


# speedup vs baseline: 2.6112x; 1.1187x over previous
import jax
import jax.numpy as jnp
from jax import lax
from jax.experimental import pallas as pl
from jax.experimental.pallas import tpu as pltpu
from jax.experimental.pallas import tpu_sc as plsc

N = 10000
E = 320000
D = 128
NP = 10240
RPT = NP // 16
HR = NP // 128
HRT = HR // 16
EPT = E // 16
CE = 80
EPC = E // 2
EPCT = EPC // 16
NCE = EPCT // CE
SB = 2000
CPS = SB // CE
CZ = 32
NCZ = RPT // CZ


def _rsqrt16(d):
    ii = lax.bitcast_convert_type(d, jnp.int32)
    ii = jnp.int32(0x5F3759DF) - (ii >> 1)
    y = lax.bitcast_convert_type(ii, jnp.float32)
    half = jnp.float32(0.5) * d
    y = y * (jnp.float32(1.5) - half * y * y)
    y = y * (jnp.float32(1.5) - half * y * y)
    y = y * (jnp.float32(1.5) - half * y * y)
    return y


def _main_body(x_hbm, src_hbm, dst_hbm, part_hbm, xs_hbm,
               deg_sh, acc_sh, zdeg, iden, dbuf, ibuf, xrow,
               ssrcb, sdstb, sidxa, didxa, sidxb, didxb,
               rowsa, rowsb, abuf, xrowb, abufb,
               sema, semb, ssema, ssemb, semz, semra, semrb, semwa, semwb):
    hist = rowsa
    zacc = abuf
    c = lax.axis_index("c")
    s = lax.axis_index("s")
    f0 = jnp.float32(0.0)

    def fill_zacc(i, _):
        for k in range(D // 16):
            zacc[i, pl.ds(k * 16, 16)] = jnp.full((16,), f0)
        return _
    lax.fori_loop(0, CZ, fill_zacc, None)

    def fill_zdeg(i, _):
        for k in range(D // 16):
            zdeg[i, pl.ds(k * 16, 16)] = jnp.full((16,), f0)
        return _
    lax.fori_loop(0, HRT, fill_zdeg, None)

    def fill_iden(k, _):
        iden[pl.ds(k * 16, 16)] = lax.iota(jnp.int32, 16) + k * 16
        return _
    lax.fori_loop(0, HR // 16, fill_iden, None)

    def fill_hist(i, _):
        for k in range(D // 16):
            hist[i, pl.ds(k * 16, 16)] = jnp.full((16,), f0)
        return _
    lax.fori_loop(0, HR, fill_hist, None)

    pltpu.async_copy(zdeg, deg_sh.at[pl.ds(s * HRT, HRT), :], semz)

    def zero_acc(j, _):
        pltpu.async_copy(zacc, acc_sh.at[pl.ds(s * RPT + j * CZ, CZ), :], semz)
        return _
    lax.fori_loop(0, NCZ, zero_acc, None)
    pltpu.make_async_copy(zdeg, deg_sh.at[pl.ds(s * HRT, HRT), :], semz).wait()

    def zero_wait(j, _):
        pltpu.make_async_copy(zacc, acc_sh.at[pl.ds(s * RPT + j * CZ, CZ), :],
                              semz).wait()
        return _
    lax.fori_loop(0, NCZ, zero_wait, None)
    plsc.subcore_barrier()

    one16 = jnp.full((16,), jnp.float32(1.0))

    NDB = EPT // SB

    def deg_load(q, bufp, semp):
        pltpu.async_copy(src_hbm.at[pl.ds(s * EPT + q * SB, SB)], bufp, semp)

    def deg_hist(q, bufp, semp):
        pltpu.make_async_copy(src_hbm.at[pl.ds(s * EPT + q * SB, SB)],
                              bufp, semp).wait()

        def deg_step(g, _):
            n = bufp[pl.ds(g * 16, 16)]
            plsc.addupdate_scatter(hist, [n >> 7, n & 127], one16)
            return _
        lax.fori_loop(0, SB // 16, deg_step, None)

    deg_load(0, ssrcb, sema)

    def deg_pair(t, _):
        deg_load(2 * t + 1, sdstb, semb)
        deg_hist(2 * t, ssrcb, sema)

        @pl.when(2 * t + 2 < NDB)
        def _():
            deg_load(2 * t + 2, ssrcb, sema)
        deg_hist(2 * t + 1, sdstb, semb)
        return _
    lax.fori_loop(0, NDB // 2, deg_pair, None)
    pltpu.sync_copy(hist, deg_sh.at[iden], add=True)
    plsc.subcore_barrier()

    pltpu.sync_copy(deg_sh.at[pl.ds(s * HRT, HRT), :], dbuf)

    def inv_step(i, _):
        r = i // 8
        k = i % 8
        d = jnp.maximum(dbuf[r, pl.ds(k * 16, 16)], jnp.float32(1.0))
        ibuf[pl.ds(i * 16, 16)] = _rsqrt16(d)
        return _
    lax.fori_loop(0, RPT // 16, inv_step, None)

    def scale_rows(j, bufp):
        def srow(i, _):
            idxv = jnp.zeros((16,), jnp.int32) + (j * CZ + i)
            sc = plsc.load_gather(ibuf, [idxv])
            for k in range(D // 16):
                bufp[i, pl.ds(k * 16, 16)] = bufp[i, pl.ds(k * 16, 16)] * sc
            return _
        lax.fori_loop(0, CZ, srow, None)

    def pre_rd(j, bufp, semp):
        return pltpu.make_async_copy(
            x_hbm.at[pl.ds(s * RPT + j * CZ, CZ), :], bufp, semp)

    def pre_wr(j, bufp, semp):
        return pltpu.make_async_copy(
            bufp, xs_hbm.at[pl.ds(c * NP + s * RPT + j * CZ, CZ), :], semp)

    pltpu.async_copy(x_hbm.at[pl.ds(s * RPT, CZ), :], xrow, semra)

    def pre_pair(t, _):
        j1 = 2 * t
        j2 = 2 * t + 1

        @pl.when(t > 0)
        def _():
            pre_wr(j2 - 2, xrowb, semwb).wait()
        pltpu.async_copy(x_hbm.at[pl.ds(s * RPT + j2 * CZ, CZ), :],
                         xrowb, semrb)
        pre_rd(j1, xrow, semra).wait()
        scale_rows(j1, xrow)
        pltpu.async_copy(xrow,
                         xs_hbm.at[pl.ds(c * NP + s * RPT + j1 * CZ, CZ), :],
                         semwa)

        @pl.when(t < NCZ // 2 - 1)
        def _():
            pre_wr(j1, xrow, semwa).wait()
            pltpu.async_copy(x_hbm.at[pl.ds(s * RPT + (j1 + 2) * CZ, CZ), :],
                             xrow, semra)
        pre_rd(j2, xrowb, semrb).wait()
        scale_rows(j2, xrowb)
        pltpu.async_copy(xrowb,
                         xs_hbm.at[pl.ds(c * NP + s * RPT + j2 * CZ, CZ), :],
                         semwb)
        return _
    lax.fori_loop(0, NCZ // 2, pre_pair, None)
    pre_wr(NCZ - 2, xrow, semwa).wait()
    pre_wr(NCZ - 1, xrowb, semwb).wait()
    plsc.subcore_barrier()

    base = c * NP

    def handle(j, sidxp, didxp, rowsp, semp, ssemp):
        @pl.when(j >= 2)
        def _():
            pltpu.make_async_copy(rowsp, acc_sh.at[didxp], ssemp).wait()

        @pl.when(j % CPS == 0)
        def _():
            e0 = c * EPC + s * EPCT + (j // CPS) * SB
            pltpu.sync_copy(src_hbm.at[pl.ds(e0, SB)], ssrcb)
            pltpu.sync_copy(dst_hbm.at[pl.ds(e0, SB)], sdstb)

        off = (j % CPS) * CE
        for k in range(CE // 16):
            sidxp[pl.ds(k * 16, 16)] = ssrcb[pl.ds(off + k * 16, 16)] + base
            didxp[pl.ds(k * 16, 16)] = sdstb[pl.ds(off + k * 16, 16)]
        pltpu.async_copy(xs_hbm.at[sidxp], rowsp, semp)

    def finish(sidxp, didxp, rowsp, semp, ssemp):
        pltpu.make_async_copy(xs_hbm.at[sidxp], rowsp, semp).wait()
        pltpu.async_copy(rowsp, acc_sh.at[didxp], ssemp, add=True)

    handle(0, sidxa, didxa, rowsa, sema, ssema)

    def edge_pair(t, _):
        handle(2 * t + 1, sidxb, didxb, rowsb, semb, ssemb)
        finish(sidxa, didxa, rowsa, sema, ssema)
        handle(2 * t + 2, sidxa, didxa, rowsa, sema, ssema)
        finish(sidxb, didxb, rowsb, semb, ssemb)
        return _
    lax.fori_loop(0, (NCE - 1) // 2, edge_pair, None)
    if NCE % 2 == 1:
        finish(sidxa, didxa, rowsa, sema, ssema)
    else:
        handle(NCE - 1, sidxb, didxb, rowsb, semb, ssemb)
        finish(sidxa, didxa, rowsa, sema, ssema)
        finish(sidxb, didxb, rowsb, semb, ssemb)
    pltpu.make_async_copy(rowsb, acc_sh.at[didxb], ssemb).wait()
    pltpu.make_async_copy(rowsa, acc_sh.at[didxa], ssema).wait()
    plsc.subcore_barrier()

    def dr_rd(j, bufp, semp):
        return pltpu.make_async_copy(
            acc_sh.at[pl.ds(s * RPT + j * CZ, CZ), :], bufp, semp)

    def dr_wr(j, bufp, semp):
        return pltpu.make_async_copy(
            bufp, part_hbm.at[c, pl.ds(s * RPT + j * CZ, CZ), :], semp)

    def scale_rows2(j, bufp):
        def srow(i, _):
            idxv = jnp.zeros((16,), jnp.int32) + (j * CZ + i)
            sc = plsc.load_gather(ibuf, [idxv])
            for k in range(D // 16):
                bufp[i, pl.ds(k * 16, 16)] = bufp[i, pl.ds(k * 16, 16)] * sc
            return _
        lax.fori_loop(0, CZ, srow, None)

    pltpu.async_copy(acc_sh.at[pl.ds(s * RPT, CZ), :], abuf, semra)

    def dr_pair(t, _):
        j1 = 2 * t
        j2 = 2 * t + 1

        @pl.when(t > 0)
        def _():
            dr_wr(j2 - 2, abufb, semwb).wait()
        pltpu.async_copy(acc_sh.at[pl.ds(s * RPT + j2 * CZ, CZ), :],
                         abufb, semrb)
        dr_rd(j1, abuf, semra).wait()
        scale_rows2(j1, abuf)
        pltpu.async_copy(abuf, part_hbm.at[c, pl.ds(s * RPT + j1 * CZ, CZ), :],
                         semwa)

        @pl.when(t < NCZ // 2 - 1)
        def _():
            dr_wr(j1, abuf, semwa).wait()
            pltpu.async_copy(acc_sh.at[pl.ds(s * RPT + (j1 + 2) * CZ, CZ), :],
                             abuf, semra)
        dr_rd(j2, abufb, semrb).wait()
        scale_rows2(j2, abufb)
        pltpu.async_copy(abufb, part_hbm.at[c, pl.ds(s * RPT + j2 * CZ, CZ), :],
                         semwb)
        return _
    lax.fori_loop(0, NCZ // 2, dr_pair, None)
    dr_wr(NCZ - 2, abuf, semwa).wait()
    dr_wr(NCZ - 1, abufb, semwb).wait()


def _sum_tc_body(p_ref, o_ref):
    o_ref[...] = p_ref[0] + p_ref[1]


@jax.jit
def _gcn_sc(x_pad, src, dst):
    mesh = plsc.VectorSubcoreMesh(core_axis_name="c", subcore_axis_name="s")
    ka = pl.kernel(
        _main_body,
        out_type=(
            jax.ShapeDtypeStruct((2, NP, D), jnp.float32),
            jax.ShapeDtypeStruct((2 * NP, D), jnp.float32),
        ),
        mesh=mesh,
        compiler_params=pltpu.CompilerParams(needs_layout_passes=False),
        scratch_types=[
            pltpu.VMEM_SHARED((HR, D), jnp.float32),
            pltpu.VMEM_SHARED((NP, D), jnp.float32),
            pltpu.VMEM((HRT, D), jnp.float32),
            pltpu.VMEM((HR,), jnp.int32),
            pltpu.VMEM((HRT, D), jnp.float32),
            pltpu.VMEM((RPT,), jnp.float32),
            pltpu.VMEM((CZ, D), jnp.float32),
            pltpu.VMEM((SB,), jnp.int32),
            pltpu.VMEM((SB,), jnp.int32),
            pltpu.VMEM((CE,), jnp.int32),
            pltpu.VMEM((CE,), jnp.int32),
            pltpu.VMEM((CE,), jnp.int32),
            pltpu.VMEM((CE,), jnp.int32),
            pltpu.VMEM((CE, D), jnp.float32),
            pltpu.VMEM((CE, D), jnp.float32),
            pltpu.VMEM((CZ, D), jnp.float32),
            pltpu.VMEM((CZ, D), jnp.float32),
            pltpu.VMEM((CZ, D), jnp.float32),
            pltpu.SemaphoreType.DMA,
            pltpu.SemaphoreType.DMA,
            pltpu.SemaphoreType.DMA,
            pltpu.SemaphoreType.DMA,
            pltpu.SemaphoreType.DMA,
            pltpu.SemaphoreType.DMA,
            pltpu.SemaphoreType.DMA,
            pltpu.SemaphoreType.DMA,
            pltpu.SemaphoreType.DMA,
        ],
    )
    part, _ = ka(x_pad, src, dst)
    return pl.pallas_call(
        _sum_tc_body,
        grid=(NP // 512,),
        in_specs=[pl.BlockSpec((2, 512, D), lambda i: (0, i, 0))],
        out_specs=pl.BlockSpec((512, D), lambda i: (i, 0)),
        out_shape=jax.ShapeDtypeStruct((NP, D), jnp.float32),
    )(part)


def kernel(x, edge_index):
    src = edge_index[0].astype(jnp.int32)
    dst = edge_index[1].astype(jnp.int32)
    x_pad = jnp.pad(x, ((0, NP - N), (0, 0)))
    out = _gcn_sc(x_pad, src, dst)
    return out[:N]


# speedup vs baseline: 2.6747x; 1.0243x over previous
import jax
import jax.numpy as jnp
from jax import lax
from jax.experimental import pallas as pl
from jax.experimental.pallas import tpu as pltpu
from jax.experimental.pallas import tpu_sc as plsc

N = 10000
E = 320000
D = 128
NP = 10240
RPT = NP // 16
HR = NP // 128
HRT = HR // 16
EPT = E // 16
CE = 80
EPC = E // 2
EPCT = EPC // 16
NCE = EPCT // CE
SB = 2000
CPS = SB // CE
CZ = 32
NCZ = RPT // CZ


def _rsqrt16(d):
    ii = lax.bitcast_convert_type(d, jnp.int32)
    ii = jnp.int32(0x5F3759DF) - (ii >> 1)
    y = lax.bitcast_convert_type(ii, jnp.float32)
    half = jnp.float32(0.5) * d
    y = y * (jnp.float32(1.5) - half * y * y)
    y = y * (jnp.float32(1.5) - half * y * y)
    y = y * (jnp.float32(1.5) - half * y * y)
    return y


def _main_body(x_hbm, src_hbm, dst_hbm, part_hbm, xs_hbm,
               deg_sh, acc_sh, zdeg, iden, dbuf, ibuf, xrow,
               ssrcb, sdstb, sidxa, didxa, sidxb, didxb,
               rowsa, rowsb, abuf, xrowb, abufb,
               sema, semb, ssema, ssemb, semz, semra, semrb, semwa, semwb):
    hist = rowsa
    zacc = abuf
    c = lax.axis_index("c")
    s = lax.axis_index("s")
    f0 = jnp.float32(0.0)

    def fill_zacc(i, _):
        for k in range(D // 16):
            zacc[i, pl.ds(k * 16, 16)] = jnp.full((16,), f0)
        return _
    lax.fori_loop(0, CZ, fill_zacc, None)

    def fill_zdeg(i, _):
        for k in range(D // 16):
            zdeg[i, pl.ds(k * 16, 16)] = jnp.full((16,), f0)
        return _
    lax.fori_loop(0, HRT, fill_zdeg, None)

    def fill_iden(k, _):
        iden[pl.ds(k * 16, 16)] = lax.iota(jnp.int32, 16) + k * 16
        return _
    lax.fori_loop(0, HR // 16, fill_iden, None)

    def fill_hist(i, _):
        for k in range(D // 16):
            hist[i, pl.ds(k * 16, 16)] = jnp.full((16,), f0)
        return _
    lax.fori_loop(0, HR, fill_hist, None)

    pltpu.async_copy(zdeg, deg_sh.at[pl.ds(s * HRT, HRT), :], semz)

    def zero_acc(j, _):
        pltpu.async_copy(zacc, acc_sh.at[pl.ds(s * RPT + j * CZ, CZ), :], semz)
        return _
    lax.fori_loop(0, NCZ, zero_acc, None)
    one16 = jnp.full((16,), jnp.float32(1.0))

    NDB = EPT // SB

    def deg_load(q, bufp, semp):
        pltpu.async_copy(src_hbm.at[pl.ds(s * EPT + q * SB, SB)], bufp, semp)

    def deg_hist(q, bufp, semp):
        pltpu.make_async_copy(src_hbm.at[pl.ds(s * EPT + q * SB, SB)],
                              bufp, semp).wait()

        def deg_step(g, _):
            n = bufp[pl.ds(g * 16, 16)]
            plsc.addupdate_scatter(hist, [n >> 7, n & 127], one16)
            return _
        lax.fori_loop(0, SB // 16, deg_step, None)

    deg_load(0, ssrcb, sema)

    def deg_pair(t, _):
        deg_load(2 * t + 1, sdstb, semb)
        deg_hist(2 * t, ssrcb, sema)

        @pl.when(2 * t + 2 < NDB)
        def _():
            deg_load(2 * t + 2, ssrcb, sema)
        deg_hist(2 * t + 1, sdstb, semb)
        return _
    lax.fori_loop(0, NDB // 2, deg_pair, None)
    pltpu.make_async_copy(zdeg, deg_sh.at[pl.ds(s * HRT, HRT), :], semz).wait()

    def zero_wait(j, _):
        pltpu.make_async_copy(zacc, acc_sh.at[pl.ds(s * RPT + j * CZ, CZ), :],
                              semz).wait()
        return _
    lax.fori_loop(0, NCZ, zero_wait, None)
    plsc.subcore_barrier()
    pltpu.sync_copy(hist, deg_sh.at[iden], add=True)
    plsc.subcore_barrier()

    pltpu.sync_copy(deg_sh.at[pl.ds(s * HRT, HRT), :], dbuf)

    def inv_step(i, _):
        r = i // 8
        k = i % 8
        d = jnp.maximum(dbuf[r, pl.ds(k * 16, 16)], jnp.float32(1.0))
        ibuf[pl.ds(i * 16, 16)] = _rsqrt16(d)
        return _
    lax.fori_loop(0, RPT // 16, inv_step, None)

    def scale_rows(j, bufp):
        def srow(i, _):
            idxv = jnp.zeros((16,), jnp.int32) + (j * CZ + i)
            sc = plsc.load_gather(ibuf, [idxv])
            for k in range(D // 16):
                bufp[i, pl.ds(k * 16, 16)] = bufp[i, pl.ds(k * 16, 16)] * sc
            return _
        lax.fori_loop(0, CZ, srow, None)

    def pre_rd(j, bufp, semp):
        return pltpu.make_async_copy(
            x_hbm.at[pl.ds(s * RPT + j * CZ, CZ), :], bufp, semp)

    def pre_wr(j, bufp, semp):
        return pltpu.make_async_copy(
            bufp, xs_hbm.at[pl.ds(c * NP + s * RPT + j * CZ, CZ), :], semp)

    pltpu.async_copy(x_hbm.at[pl.ds(s * RPT, CZ), :], xrow, semra)

    def pre_pair(t, _):
        j1 = 2 * t
        j2 = 2 * t + 1

        @pl.when(t > 0)
        def _():
            pre_wr(j2 - 2, xrowb, semwb).wait()
        pltpu.async_copy(x_hbm.at[pl.ds(s * RPT + j2 * CZ, CZ), :],
                         xrowb, semrb)
        pre_rd(j1, xrow, semra).wait()
        scale_rows(j1, xrow)
        pltpu.async_copy(xrow,
                         xs_hbm.at[pl.ds(c * NP + s * RPT + j1 * CZ, CZ), :],
                         semwa)

        @pl.when(t < NCZ // 2 - 1)
        def _():
            pre_wr(j1, xrow, semwa).wait()
            pltpu.async_copy(x_hbm.at[pl.ds(s * RPT + (j1 + 2) * CZ, CZ), :],
                             xrow, semra)
        pre_rd(j2, xrowb, semrb).wait()
        scale_rows(j2, xrowb)
        pltpu.async_copy(xrowb,
                         xs_hbm.at[pl.ds(c * NP + s * RPT + j2 * CZ, CZ), :],
                         semwb)
        return _
    lax.fori_loop(0, NCZ // 2, pre_pair, None)
    pre_wr(NCZ - 2, xrow, semwa).wait()
    pre_wr(NCZ - 1, xrowb, semwb).wait()
    plsc.subcore_barrier()

    base = c * NP

    def handle(j, sidxp, didxp, rowsp, semp, ssemp):
        @pl.when(j >= 2)
        def _():
            pltpu.make_async_copy(rowsp, acc_sh.at[didxp], ssemp).wait()

        @pl.when(j % CPS == 0)
        def _():
            e0 = c * EPC + s * EPCT + (j // CPS) * SB
            pltpu.sync_copy(src_hbm.at[pl.ds(e0, SB)], ssrcb)
            pltpu.sync_copy(dst_hbm.at[pl.ds(e0, SB)], sdstb)

        off = (j % CPS) * CE
        for k in range(CE // 16):
            sidxp[pl.ds(k * 16, 16)] = ssrcb[pl.ds(off + k * 16, 16)] + base
            didxp[pl.ds(k * 16, 16)] = sdstb[pl.ds(off + k * 16, 16)]
        pltpu.async_copy(xs_hbm.at[sidxp], rowsp, semp)

    def finish(sidxp, didxp, rowsp, semp, ssemp):
        pltpu.make_async_copy(xs_hbm.at[sidxp], rowsp, semp).wait()
        pltpu.async_copy(rowsp, acc_sh.at[didxp], ssemp, add=True)

    handle(0, sidxa, didxa, rowsa, sema, ssema)

    def edge_pair(t, _):
        handle(2 * t + 1, sidxb, didxb, rowsb, semb, ssemb)
        finish(sidxa, didxa, rowsa, sema, ssema)
        handle(2 * t + 2, sidxa, didxa, rowsa, sema, ssema)
        finish(sidxb, didxb, rowsb, semb, ssemb)
        return _
    lax.fori_loop(0, (NCE - 1) // 2, edge_pair, None)
    if NCE % 2 == 1:
        finish(sidxa, didxa, rowsa, sema, ssema)
    else:
        handle(NCE - 1, sidxb, didxb, rowsb, semb, ssemb)
        finish(sidxa, didxa, rowsa, sema, ssema)
        finish(sidxb, didxb, rowsb, semb, ssemb)
    pltpu.make_async_copy(rowsb, acc_sh.at[didxb], ssemb).wait()
    pltpu.make_async_copy(rowsa, acc_sh.at[didxa], ssema).wait()
    plsc.subcore_barrier()

    def dr_rd(j, bufp, semp):
        return pltpu.make_async_copy(
            acc_sh.at[pl.ds(s * RPT + j * CZ, CZ), :], bufp, semp)

    def dr_wr(j, bufp, semp):
        return pltpu.make_async_copy(
            bufp, part_hbm.at[c, pl.ds(s * RPT + j * CZ, CZ), :], semp)

    def scale_rows2(j, bufp):
        def srow(i, _):
            idxv = jnp.zeros((16,), jnp.int32) + (j * CZ + i)
            sc = plsc.load_gather(ibuf, [idxv])
            for k in range(D // 16):
                bufp[i, pl.ds(k * 16, 16)] = bufp[i, pl.ds(k * 16, 16)] * sc
            return _
        lax.fori_loop(0, CZ, srow, None)

    pltpu.async_copy(acc_sh.at[pl.ds(s * RPT, CZ), :], abuf, semra)

    def dr_pair(t, _):
        j1 = 2 * t
        j2 = 2 * t + 1

        @pl.when(t > 0)
        def _():
            dr_wr(j2 - 2, abufb, semwb).wait()
        pltpu.async_copy(acc_sh.at[pl.ds(s * RPT + j2 * CZ, CZ), :],
                         abufb, semrb)
        dr_rd(j1, abuf, semra).wait()
        scale_rows2(j1, abuf)
        pltpu.async_copy(abuf, part_hbm.at[c, pl.ds(s * RPT + j1 * CZ, CZ), :],
                         semwa)

        @pl.when(t < NCZ // 2 - 1)
        def _():
            dr_wr(j1, abuf, semwa).wait()
            pltpu.async_copy(acc_sh.at[pl.ds(s * RPT + (j1 + 2) * CZ, CZ), :],
                             abuf, semra)
        dr_rd(j2, abufb, semrb).wait()
        scale_rows2(j2, abufb)
        pltpu.async_copy(abufb, part_hbm.at[c, pl.ds(s * RPT + j2 * CZ, CZ), :],
                         semwb)
        return _
    lax.fori_loop(0, NCZ // 2, dr_pair, None)
    dr_wr(NCZ - 2, abuf, semwa).wait()
    dr_wr(NCZ - 1, abufb, semwb).wait()


def _sum_tc_body(p_ref, o_ref):
    o_ref[...] = p_ref[0] + p_ref[1]


@jax.jit
def _gcn_sc(x_pad, src, dst):
    mesh = plsc.VectorSubcoreMesh(core_axis_name="c", subcore_axis_name="s")
    ka = pl.kernel(
        _main_body,
        out_type=(
            jax.ShapeDtypeStruct((2, NP, D), jnp.float32),
            jax.ShapeDtypeStruct((2 * NP, D), jnp.float32),
        ),
        mesh=mesh,
        compiler_params=pltpu.CompilerParams(needs_layout_passes=False),
        scratch_types=[
            pltpu.VMEM_SHARED((HR, D), jnp.float32),
            pltpu.VMEM_SHARED((NP, D), jnp.float32),
            pltpu.VMEM((HRT, D), jnp.float32),
            pltpu.VMEM((HR,), jnp.int32),
            pltpu.VMEM((HRT, D), jnp.float32),
            pltpu.VMEM((RPT,), jnp.float32),
            pltpu.VMEM((CZ, D), jnp.float32),
            pltpu.VMEM((SB,), jnp.int32),
            pltpu.VMEM((SB,), jnp.int32),
            pltpu.VMEM((CE,), jnp.int32),
            pltpu.VMEM((CE,), jnp.int32),
            pltpu.VMEM((CE,), jnp.int32),
            pltpu.VMEM((CE,), jnp.int32),
            pltpu.VMEM((CE, D), jnp.float32),
            pltpu.VMEM((CE, D), jnp.float32),
            pltpu.VMEM((CZ, D), jnp.float32),
            pltpu.VMEM((CZ, D), jnp.float32),
            pltpu.VMEM((CZ, D), jnp.float32),
            pltpu.SemaphoreType.DMA,
            pltpu.SemaphoreType.DMA,
            pltpu.SemaphoreType.DMA,
            pltpu.SemaphoreType.DMA,
            pltpu.SemaphoreType.DMA,
            pltpu.SemaphoreType.DMA,
            pltpu.SemaphoreType.DMA,
            pltpu.SemaphoreType.DMA,
            pltpu.SemaphoreType.DMA,
        ],
    )
    part, _ = ka(x_pad, src, dst)
    return pl.pallas_call(
        _sum_tc_body,
        grid=(N // 400,),
        in_specs=[pl.BlockSpec((2, 400, D), lambda i: (0, i, 0))],
        out_specs=pl.BlockSpec((400, D), lambda i: (i, 0)),
        out_shape=jax.ShapeDtypeStruct((N, D), jnp.float32),
    )(part)


def kernel(x, edge_index):
    src = edge_index[0].astype(jnp.int32)
    dst = edge_index[1].astype(jnp.int32)
    x_pad = jnp.pad(x, ((0, NP - N), (0, 0)))
    return _gcn_sc(x_pad, src, dst)
